# baseline (speedup 1.0000x reference)
import functools
import jax
import jax.numpy as jnp
from jax.experimental import pallas as pl
from jax.experimental.pallas import tpu as pltpu

_N = 10000
_BI1 = 200
_BI2 = 1000
_F8MAX = 448.0


def _layer1_body(adj_ref, x_ref, w1_ref, b_ref, wn_ref, o_ref, adj8_ref):
    a = adj_ref[...]
    adj8_ref[...] = a.astype(jnp.float8_e4m3fn)
    z0 = jnp.dot(x_ref[...], w1_ref[...], preferred_element_type=jnp.float32)
    h = jnp.dot(a, z0, preferred_element_type=jnp.float32)
    h = jnp.maximum(h + b_ref[...], 0.0)
    o_ref[...] = jnp.dot(h, wn_ref[...],
                         preferred_element_type=jnp.float32).astype(jnp.bfloat16)


def _spmm_layer1(adj, x, w1, b, wn):
    n = adj.shape[0]
    grid = (n // _BI1,)
    return pl.pallas_call(
        _layer1_body,
        grid=grid,
        out_shape=[
            jax.ShapeDtypeStruct((n, wn.shape[1]), jnp.bfloat16),
            jax.ShapeDtypeStruct((n, n), jnp.float8_e4m3fn),
        ],
        in_specs=[
            pl.BlockSpec((_BI1, n), lambda i: (i, 0)),
            pl.BlockSpec(x.shape, lambda i: (0, 0)),
            pl.BlockSpec(w1.shape, lambda i: (0, 0)),
            pl.BlockSpec((1, b.shape[1]), lambda i: (0, 0)),
            pl.BlockSpec(wn.shape, lambda i: (0, 0)),
        ],
        out_specs=[
            pl.BlockSpec((_BI1, wn.shape[1]), lambda i: (i, 0)),
            pl.BlockSpec((_BI1, n), lambda i: (i, 0)),
        ],
        compiler_params=pltpu.CompilerParams(
            dimension_semantics=("arbitrary",),
        ),
    )(adj, x, w1, b, wn)


def _quant_two_plane(z):
    s = jnp.max(jnp.abs(z), axis=0, keepdims=True) / _F8MAX
    s = jnp.maximum(s, 1e-30)
    zs = z / s
    zh = zs.astype(jnp.float8_e4m3fn)
    zl = ((zs - zh.astype(jnp.float32)) * 16.0).astype(jnp.float8_e4m3fn)
    return jnp.concatenate([zh, zl], axis=1), s


def _layers23_body(adj_ref, z1_ref, b2_ref, b3_ref, w3_ref, wl_ref, bl_ref,
                   o_ref, zq_ref, s_ref, z2_ref):
    stage = pl.program_id(0)
    i = pl.program_id(1)
    k = z1_ref.shape[1]

    @pl.when(jnp.logical_and(stage == 0, i == 0))
    def _():
        zq, s = _quant_two_plane(z1_ref[...].astype(jnp.float32))
        zq_ref[...] = zq
        s_ref[...] = s

    @pl.when(jnp.logical_and(stage == 1, i == 0))
    def _():
        zq, s = _quant_two_plane(z2_ref[...])
        zq_ref[...] = zq
        s_ref[...] = s

    b = jnp.where(stage == 0, b2_ref[...], b3_ref[...])
    d = jnp.dot(adj_ref[...], zq_ref[...], preferred_element_type=jnp.float32)
    h = jnp.maximum(
        s_ref[...] * (d[:, :k] + d[:, k:] * (1.0 / 16.0)) + b, 0.0)

    @pl.when(stage == 0)
    def _():
        z2_ref[pl.ds(i * _BI2, _BI2), :] = jnp.dot(
            h, w3_ref[...], preferred_element_type=jnp.float32)

    @pl.when(stage == 1)
    def _():
        logits = jnp.dot(h, wl_ref[...],
                         preferred_element_type=jnp.float32) + bl_ref[...]
        m = jnp.max(logits, axis=1, keepdims=True)
        sh = logits - m
        lse = jnp.log(jnp.sum(jnp.exp(sh), axis=1, keepdims=True))
        o_ref[...] = sh - lse


def _spmm_layers23(adj8, z1, b2, b3, w3, wl, bl):
    n = adj8.shape[0]
    k = z1.shape[1]
    nclass = wl.shape[1]
    grid = (2, n // _BI2)
    return pl.pallas_call(
        _layers23_body,
        grid=grid,
        out_shape=jax.ShapeDtypeStruct((n, nclass), jnp.float32),
        in_specs=[
            pl.BlockSpec((_BI2, n), lambda s, i: (i, 0)),
            pl.BlockSpec((n, k), lambda s, i: (0, 0)),
            pl.BlockSpec((1, k), lambda s, i: (0, 0)),
            pl.BlockSpec((1, k), lambda s, i: (0, 0)),
            pl.BlockSpec(w3.shape, lambda s, i: (0, 0)),
            pl.BlockSpec(wl.shape, lambda s, i: (0, 0)),
            pl.BlockSpec((1, nclass), lambda s, i: (0, 0)),
        ],
        out_specs=pl.BlockSpec((_BI2, nclass), lambda s, i: (i, 0)),
        scratch_shapes=[
            pltpu.VMEM((n, 2 * k), jnp.float8_e4m3fn),
            pltpu.VMEM((1, k), jnp.float32),
            pltpu.VMEM((n, k), jnp.float32),
        ],
        compiler_params=pltpu.CompilerParams(
            dimension_semantics=("arbitrary", "arbitrary"),
        ),
    )(adj8, z1, b2, b3, w3, wl, bl)


def kernel(x, adj, W1, b1, W2, b2, W3, b3, Wl, bl):
    b1 = b1.reshape(1, -1)
    b2 = b2.reshape(1, -1)
    b3 = b3.reshape(1, -1)
    bl = bl.reshape(1, -1)
    z1, adj8 = _spmm_layer1(adj, x, W1, b1, W2)
    out = _spmm_layers23(adj8, z1, b2, b3, W3, Wl, bl)
    return out


# speedup vs baseline: 1.0637x; 1.0637x over previous
import functools
import jax
import jax.numpy as jnp
from jax.experimental import pallas as pl
from jax.experimental.pallas import tpu as pltpu

_N = 10000
_BI1 = 400
_BI2 = 1000
_F8MAX = 448.0


def _layer1_body(adj_ref, x_ref, w1_ref, b_ref, wn_ref, o_ref, adj8_ref):
    a = adj_ref[...]
    adj8_ref[...] = a.astype(jnp.float8_e4m3fn)
    z0 = jnp.dot(x_ref[...], w1_ref[...], preferred_element_type=jnp.float32)
    h = jnp.dot(a, z0, preferred_element_type=jnp.float32)
    h = jnp.maximum(h + b_ref[...], 0.0)
    o_ref[...] = jnp.dot(h, wn_ref[...],
                         preferred_element_type=jnp.float32).astype(jnp.bfloat16)


def _spmm_layer1(adj, x, w1, b, wn):
    n = adj.shape[0]
    grid = (n // _BI1,)
    return pl.pallas_call(
        _layer1_body,
        grid=grid,
        out_shape=[
            jax.ShapeDtypeStruct((n, wn.shape[1]), jnp.bfloat16),
            jax.ShapeDtypeStruct((n, n), jnp.float8_e4m3fn),
        ],
        in_specs=[
            pl.BlockSpec((_BI1, n), lambda i: (i, 0)),
            pl.BlockSpec(x.shape, lambda i: (0, 0)),
            pl.BlockSpec(w1.shape, lambda i: (0, 0)),
            pl.BlockSpec((1, b.shape[1]), lambda i: (0, 0)),
            pl.BlockSpec(wn.shape, lambda i: (0, 0)),
        ],
        out_specs=[
            pl.BlockSpec((_BI1, wn.shape[1]), lambda i: (i, 0)),
            pl.BlockSpec((_BI1, n), lambda i: (i, 0)),
        ],
        compiler_params=pltpu.CompilerParams(
            dimension_semantics=("arbitrary",),
        ),
    )(adj, x, w1, b, wn)


def _quant_two_plane(z):
    s = jnp.max(jnp.abs(z), axis=0, keepdims=True) / _F8MAX
    s = jnp.maximum(s, 1e-30)
    zs = z / s
    zh = zs.astype(jnp.float8_e4m3fn)
    zl = ((zs - zh.astype(jnp.float32)) * 16.0).astype(jnp.float8_e4m3fn)
    return jnp.concatenate([zh, zl], axis=1), s


def _layers23_body(adj_ref, z1_ref, b2_ref, b3_ref, w3_ref, wl_ref, bl_ref,
                   o_ref, zq_ref, s_ref, z2_ref):
    stage = pl.program_id(0)
    i = pl.program_id(1)
    k = z1_ref.shape[1]

    @pl.when(jnp.logical_and(stage == 0, i == 0))
    def _():
        zq, s = _quant_two_plane(z1_ref[...].astype(jnp.float32))
        zq_ref[...] = zq
        s_ref[...] = s

    @pl.when(jnp.logical_and(stage == 1, i == 0))
    def _():
        zq, s = _quant_two_plane(z2_ref[...])
        zq_ref[...] = zq
        s_ref[...] = s

    b = jnp.where(stage == 0, b2_ref[...], b3_ref[...])
    d = jnp.dot(adj_ref[...], zq_ref[...], preferred_element_type=jnp.float32)
    h = jnp.maximum(
        s_ref[...] * (d[:, :k] + d[:, k:] * (1.0 / 16.0)) + b, 0.0)

    @pl.when(stage == 0)
    def _():
        z2_ref[pl.ds(i * _BI2, _BI2), :] = jnp.dot(
            h, w3_ref[...], preferred_element_type=jnp.float32)

    @pl.when(stage == 1)
    def _():
        logits = jnp.dot(h, wl_ref[...],
                         preferred_element_type=jnp.float32) + bl_ref[...]
        m = jnp.max(logits, axis=1, keepdims=True)
        sh = logits - m
        lse = jnp.log(jnp.sum(jnp.exp(sh), axis=1, keepdims=True))
        o_ref[...] = sh - lse


def _spmm_layers23(adj8, z1, b2, b3, w3, wl, bl):
    n = adj8.shape[0]
    k = z1.shape[1]
    nclass = wl.shape[1]
    grid = (2, n // _BI2)
    return pl.pallas_call(
        _layers23_body,
        grid=grid,
        out_shape=jax.ShapeDtypeStruct((n, nclass), jnp.float32),
        in_specs=[
            pl.BlockSpec((_BI2, n), lambda s, i: (i, 0)),
            pl.BlockSpec((n, k), lambda s, i: (0, 0)),
            pl.BlockSpec((1, k), lambda s, i: (0, 0)),
            pl.BlockSpec((1, k), lambda s, i: (0, 0)),
            pl.BlockSpec(w3.shape, lambda s, i: (0, 0)),
            pl.BlockSpec(wl.shape, lambda s, i: (0, 0)),
            pl.BlockSpec((1, nclass), lambda s, i: (0, 0)),
        ],
        out_specs=pl.BlockSpec((_BI2, nclass), lambda s, i: (i, 0)),
        scratch_shapes=[
            pltpu.VMEM((n, 2 * k), jnp.float8_e4m3fn),
            pltpu.VMEM((1, k), jnp.float32),
            pltpu.VMEM((n, k), jnp.float32),
        ],
        compiler_params=pltpu.CompilerParams(
            dimension_semantics=("arbitrary", "arbitrary"),
        ),
    )(adj8, z1, b2, b3, w3, wl, bl)


def kernel(x, adj, W1, b1, W2, b2, W3, b3, Wl, bl):
    b1 = b1.reshape(1, -1)
    b2 = b2.reshape(1, -1)
    b3 = b3.reshape(1, -1)
    bl = bl.reshape(1, -1)
    z1, adj8 = _spmm_layer1(adj, x, W1, b1, W2)
    out = _spmm_layers23(adj8, z1, b2, b3, W3, Wl, bl)
    return out
